# baseline (speedup 1.0000x reference)
import jax
import jax.numpy as jnp
from jax.experimental import pallas as pl
from jax.experimental.pallas import tpu as pltpu

N = 10000
D_IN = 256
D_OUT = 256

BM = 200
S = 2


def _fused_kernel(*refs):
    x_ref, w_ref = refs[0], refs[1]
    adj_refs = refs[2:2 + S]
    out_ref = refs[2 + S]
    h_scratch = refs[3 + S]

    @pl.when(pl.program_id(0) == 0)
    def _compute_hidden():
        h_scratch[...] = jnp.dot(x_ref[...], w_ref[...],
                                 preferred_element_type=jnp.float32)

    for j in range(S):
        acc = jnp.dot(adj_refs[j][...], h_scratch[...],
                      preferred_element_type=jnp.float32)
        out_ref[j * BM:(j + 1) * BM, :] = jnp.maximum(acc, 0.0)


def _adj_spec(j):
    return pl.BlockSpec((BM, N), lambda i, j=j: (S * i + j, 0))


@jax.jit
def kernel(x, adj, W):
    out = pl.pallas_call(
        _fused_kernel,
        grid=(N // (S * BM),),
        in_specs=[
            pl.BlockSpec((N, D_IN), lambda i: (0, 0)),
            pl.BlockSpec((D_IN, D_OUT), lambda i: (0, 0)),
        ] + [_adj_spec(j) for j in range(S)],
        out_specs=pl.BlockSpec((S * BM, D_OUT), lambda i: (i, 0)),
        out_shape=jax.ShapeDtypeStruct((N, D_OUT), jnp.float32),
        scratch_shapes=[pltpu.VMEM((N, D_OUT), jnp.float32)],
    )(x, W, *([adj] * S))

    return (out, adj)
